# baseline (speedup 1.0000x reference)
import jax
import jax.numpy as jnp
from jax import lax
from jax.experimental import pallas as pl
from jax.experimental.pallas import tpu as pltpu
from jax.experimental.pallas import tpu_sc as plsc

L = 16
NW = 32
BC = 4096
UNROLL = 8


def _make_sc_kernel(B, E, F, U_VOCAB, I_VOCAB, NSEQ):
    NCH = B // BC
    GRP = BC // L // UNROLL

    def tower(tbl, idx_hbm, plane_v, idx_v, outs, pos_v, out_hbm,
              e, row0, sem, osem):
        pltpu.sync_copy(idx_hbm, idx_v)
        pltpu.async_copy(outs[0], out_hbm.at[row0, e, pl.ds(0, BC)], osem)
        pltpu.async_copy(outs[1], out_hbm.at[row0, e, pl.ds(BC, BC)], osem)
        pltpu.async_copy(tbl.at[e, 0, :], plane_v, sem)

        def plane(f, carry):
            pltpu.make_async_copy(tbl.at[e, 0, :], plane_v, sem).wait()
            row = row0 + f
            pos_splat = plsc.load_gather(
                pos_v, [jnp.full((L,), 0, jnp.int32) + (row * E + e)])
            for c in range(NCH):
                out_c = outs[c % 2]
                pltpu.make_async_copy(
                    out_hbm.at[row0, e, pl.ds(0, BC)], out_c, osem).wait()
                @plsc.parallel_loop(0, BC, step=L, unroll=UNROLL)
                def grp(o):
                    u16 = idx_v[pl.ds(c * BC + o, L)]
                    v = plsc.load_gather(plane_v, [u16])
                    out_c[pl.ds(o, L)] = v + pos_splat
                pltpu.async_copy(out_c, out_hbm.at[row, e, pl.ds(c * BC, BC)],
                                 osem)
            @pl.when(f + 1 < F)
            def _pf():
                pltpu.async_copy(tbl.at[e, f + 1, :], plane_v, sem)
            return carry
        lax.fori_loop(0, F, plane, 0)
        pltpu.make_async_copy(out_hbm.at[row0, e, pl.ds(0, BC)], outs[0],
                              osem).wait()
        pltpu.make_async_copy(out_hbm.at[row0, e, pl.ds(0, BC)], outs[1],
                              osem).wait()

    def body(u_hbm, i_hbm, ut_hbm, it_hbm, cls_hbm, pos_hbm, out_hbm,
             plane_v, idx_v, out_a, out_b, pos_v, cls_v, sem, osem):
        wid = lax.axis_index("s") * 2 + lax.axis_index("c")
        e = wid % E
        pltpu.sync_copy(pos_hbm, pos_v)
        pltpu.sync_copy(cls_hbm, cls_v)

        @pl.when(wid < E)
        def _user():
            esplat = jnp.full((L,), 0, jnp.int32) + e
            splat = plsc.load_gather(cls_v, [esplat]) + \
                plsc.load_gather(pos_v, [esplat])
            def fill(g, carry):
                for j in range(UNROLL):
                    out_a[pl.ds(g * (L * UNROLL) + j * L, L)] = splat
                return carry
            lax.fori_loop(0, GRP, fill, 0)
            def wr(c, carry):
                pltpu.sync_copy(out_a, out_hbm.at[0, e, pl.ds(c * BC, BC)])
                return carry
            lax.fori_loop(0, NCH, wr, 0)
            tower(ut_hbm, u_hbm, plane_v, idx_v, (out_a, out_b), pos_v,
                  out_hbm, e, 1, sem, osem)

        @pl.when(wid >= E)
        def _item():
            tower(it_hbm, i_hbm, plane_v, idx_v, (out_a, out_b), pos_v,
                  out_hbm, e, 1 + F, sem, osem)

    return pl.kernel(
        body,
        out_type=jax.ShapeDtypeStruct((NSEQ, E, B), jnp.float32),
        mesh=plsc.VectorSubcoreMesh(core_axis_name="c", subcore_axis_name="s"),
        compiler_params=pltpu.CompilerParams(
            needs_layout_passes=False, use_tc_tiling_on_sc=True),
        scratch_types=[
            pltpu.VMEM((U_VOCAB,), jnp.float32),
            pltpu.VMEM((B,), jnp.int32),
            pltpu.VMEM((BC,), jnp.float32),
            pltpu.VMEM((BC,), jnp.float32),
            pltpu.VMEM((NSEQ * E,), jnp.float32),
            pltpu.VMEM((E,), jnp.float32),
            pltpu.SemaphoreType.DMA,
            pltpu.SemaphoreType.DMA,
        ],
    )


def kernel(user, item, user_tables, item_tables, cls_token, positions):
    B = user.shape[0]
    E, U_VOCAB, F = user_tables.shape
    I_VOCAB = item_tables.shape[1]
    NSEQ = 2 * F + 1
    u = user.reshape(B)
    i = item.reshape(B)
    ut = jnp.transpose(user_tables, (0, 2, 1))
    it = jnp.transpose(item_tables, (0, 2, 1))
    cls_flat = cls_token.reshape(E)
    pos_flat = positions.reshape(NSEQ * E)
    fn = _make_sc_kernel(B, E, F, U_VOCAB, I_VOCAB, NSEQ)
    out = fn(u, i, ut, it, cls_flat, pos_flat)
    return jnp.transpose(out, (2, 0, 1))
